# baseline (speedup 1.0000x reference)
import functools

import jax
import jax.numpy as jnp
from jax import lax
from jax.experimental import pallas as pl
from jax.experimental.pallas import tpu as pltpu
from jax.experimental.pallas import tpu_sc as plsc

NUM_CORES = 2
NUM_SUBCORES = 16
NUM_WORKERS = NUM_CORES * NUM_SUBCORES
G = 128
S = 26
B = 16384
D = 64
N_UNITS = S * B // G
U_PER_W = N_UNITS // NUM_WORKERS
L = 16


@jax.jit
def _sc_embed(idx3, w500):
  mesh = plsc.VectorSubcoreMesh(
      core_axis_name="c", subcore_axis_name="s",
      num_cores=NUM_CORES, num_subcores=NUM_SUBCORES)

  @functools.partial(
      pl.kernel,
      out_type=jax.ShapeDtypeStruct((S, D, B), jnp.float32),
      mesh=mesh,
      scratch_types=[
          pltpu.VMEM((U_PER_W, G), jnp.int32),
          pltpu.VMEM((2, G), jnp.int32),
          pltpu.VMEM((2, G, 128), jnp.float32),
          pltpu.VMEM((D, G), jnp.float32),
          pltpu.SemaphoreType.DMA,
          pltpu.SemaphoreType.DMA,
      ],
      compiler_params=pltpu.CompilerParams(use_tc_tiling_on_sc=True,
                                           needs_layout_passes=False),
  )
  def body(idx_hbm, tab_hbm, out_hbm, idx_v, idx2_v, rows_v, blk_v, ga, gb):
    wid = lax.axis_index("s") * NUM_CORES + lax.axis_index("c")
    pltpu.sync_copy(idx_hbm.at[wid], idx_v)
    iota = lax.iota(jnp.int32, L)
    diag = [lax.bitwise_and(iota + d, L - 1) for d in range(L)]
    fullm = iota < L
    diag128 = [d * G for d in diag]
    iota128 = iota * 128

    def prep(g, slot):
      for k in range(G // L):
        idx2_v[slot, pl.ds(k * L, L)] = (
            lax.shift_right_logical(idx_v[g, pl.ds(k * L, L)], 1))

    def fire(g, slot, sem):
      pltpu.async_copy(tab_hbm.at[idx2_v.at[slot]], rows_v.at[slot], sem)

    def drain(slot, sem):
      pltpu.make_async_copy(tab_hbm.at[idx2_v.at[slot]], rows_v.at[slot],
                            sem).wait()

    def select_store(g, slot):
      rows = rows_v.at[slot]

      @plsc.parallel_loop(0, G // L)
      def _(k):
        rowi = iota + k * L
        par64 = lax.bitwise_and(idx_v[g, pl.ds(k * L, L)], 1) * D
        for cg in range(D // L):
          cb = par64 + (cg * L)
          vs = [plsc.load_gather(rows, [rowi, cb + diag[d]], mask=fullm)
                for d in range(L)]
          for d in range(L):
            plsc.store_scatter(blk_v, [diag[d] + (cg * L), rowi], vs[d],
                               mask=fullm)
      j = wid * U_PER_W + g
      s_i = j // 128
      bb = lax.rem(j, 128)
      pltpu.sync_copy(blk_v, out_hbm.at[s_i, :, pl.ds(bb * G, G)])

    prep(0, 0)
    fire(0, 0, ga)
    prep(1, 1)
    fire(1, 1, gb)

    @pl.loop(0, U_PER_W - 2, step=2)
    def _(g):
      drain(0, ga)
      select_store(g, 0)
      prep(g + 2, 0)
      fire(g + 2, 0, ga)
      drain(1, gb)
      select_store(g + 1, 1)
      prep(g + 3, 1)
      fire(g + 3, 1, gb)

    drain(0, ga)
    select_store(U_PER_W - 2, 0)
    drain(1, gb)
    select_store(U_PER_W - 1, 1)

  return body(idx3, w500)


def kernel(input_, weight):
  idx3 = input_.astype(jnp.int32).T.reshape(NUM_WORKERS, U_PER_W, G)
  w500 = jnp.concatenate([weight[0::2], weight[1::2]], axis=1)
  out = _sc_embed(idx3, w500)
  return out.transpose(2, 0, 1)


# speedup vs baseline: 11.4505x; 11.4505x over previous
import functools

import jax
import jax.numpy as jnp
from jax import lax
from jax.experimental import pallas as pl
from jax.experimental.pallas import tpu as pltpu
from jax.experimental.pallas import tpu_sc as plsc

NUM_CORES = 2
NUM_SUBCORES = 16
NUM_WORKERS = NUM_CORES * NUM_SUBCORES
G = 128
S = 26
B = 16384
D = 64
N_UNITS = S * B // G
U_PER_W = N_UNITS // NUM_WORKERS
L = 16


@jax.jit
def _sc_embed(idx3, w500):
  mesh = plsc.VectorSubcoreMesh(
      core_axis_name="c", subcore_axis_name="s",
      num_cores=NUM_CORES, num_subcores=NUM_SUBCORES)

  @functools.partial(
      pl.kernel,
      out_type=jax.ShapeDtypeStruct((S, D, B), jnp.float32),
      mesh=mesh,
      scratch_types=[
          pltpu.VMEM((U_PER_W, G), jnp.int32),
          pltpu.VMEM((2, G), jnp.int32),
          pltpu.VMEM((2, G, 128), jnp.float32),
          pltpu.VMEM((D, G), jnp.float32),
          pltpu.SemaphoreType.DMA,
          pltpu.SemaphoreType.DMA,
      ],
      compiler_params=pltpu.CompilerParams(use_tc_tiling_on_sc=True,
                                           needs_layout_passes=False),
  )
  def body(idx_hbm, tab_hbm, out_hbm, idx_v, idx2_v, rows_v, blk_v, ga, gb):
    wid = lax.axis_index("s") * NUM_CORES + lax.axis_index("c")
    pltpu.sync_copy(idx_hbm.at[wid], idx_v)
    iota = lax.iota(jnp.int32, L)
    diag = [lax.bitwise_and(iota + d, L - 1) for d in range(L)]
    fullm = iota < L
    diag128 = [d * G for d in diag]
    iota128 = iota * 128

    def prep(g, slot):
      for k in range(G // L):
        idx2_v[slot, pl.ds(k * L, L)] = (
            lax.shift_right_logical(idx_v[g, pl.ds(k * L, L)], 1))

    def fire(g, slot, sem):
      pltpu.async_copy(tab_hbm.at[idx2_v.at[slot]], rows_v.at[slot], sem)

    def drain(slot, sem):
      pltpu.make_async_copy(tab_hbm.at[idx2_v.at[slot]], rows_v.at[slot],
                            sem).wait()

    def select_store(g, slot):
      rows = rows_v.at[slot]

      @plsc.parallel_loop(0, G // L)
      def _(k):
        rowi = iota + k * L
        par64 = lax.bitwise_and(idx_v[g, pl.ds(k * L, L)], 1) * D
        for cgp in range(D // L // 2):
          vs = []
          for cg in (2 * cgp, 2 * cgp + 1):
            cb = par64 + (cg * L)
            vs += [plsc.load_gather(rows, [rowi, cb + diag[d]], mask=fullm)
                   for d in range(L)]
          for h, cg in enumerate((2 * cgp, 2 * cgp + 1)):
            for d in range(L):
              plsc.store_scatter(blk_v, [diag[d] + (cg * L), rowi],
                                 vs[h * L + d], mask=fullm)
      j = wid * U_PER_W + g
      s_i = j // 128
      bb = lax.rem(j, 128)
      pltpu.sync_copy(blk_v, out_hbm.at[s_i, :, pl.ds(bb * G, G)])

    prep(0, 0)
    fire(0, 0, ga)
    prep(1, 1)
    fire(1, 1, gb)

    @pl.loop(0, U_PER_W - 2, step=2)
    def _(g):
      drain(0, ga)
      select_store(g, 0)
      prep(g + 2, 0)
      fire(g + 2, 0, ga)
      drain(1, gb)
      select_store(g + 1, 1)
      prep(g + 3, 1)
      fire(g + 3, 1, gb)

    drain(0, ga)
    select_store(U_PER_W - 2, 0)
    drain(1, gb)
    select_store(U_PER_W - 1, 1)

  return body(idx3, w500)


def kernel(input_, weight):
  idx3 = input_.astype(jnp.int32).T.reshape(NUM_WORKERS, U_PER_W, G)
  w500 = weight.reshape(500000, 128)
  out = _sc_embed(idx3, w500)
  return out.transpose(2, 0, 1)
